# Initial kernel scaffold; baseline (speedup 1.0000x reference)
import jax
import jax.numpy as jnp
from jax.experimental import pallas as pl


def kernel(diffusion_step, embedding, W1, b1, W2, b2):
    raise NotImplementedError("write your pallas kernel here")


# speedup vs baseline: 1.4583x; 1.4583x over previous
import functools

import jax
import jax.numpy as jnp
from jax import lax
from jax.experimental import pallas as pl
from jax.experimental.pallas import tpu as pltpu
from jax.experimental.pallas import tpu_sc as plsc

_TABLE_ROWS = 1000
_D_IN = 128
_D_HID = 512
_D_OUT = 512
_BATCH = 16384


def _mlp_table_body(emb_ref, w1_ref, b1_ref, w2_ref, b2_ref, out_ref):
    h = jnp.dot(emb_ref[...], w1_ref[...], preferred_element_type=jnp.float32)
    h = h + b1_ref[...]
    h = h * lax.logistic(h)
    o = jnp.dot(h, w2_ref[...], preferred_element_type=jnp.float32)
    o = o + b2_ref[...]
    out_ref[...] = o * lax.logistic(o)


def _mlp_table(embedding, W1, b1, W2, b2):
    return pl.pallas_call(
        _mlp_table_body,
        out_shape=jax.ShapeDtypeStruct((_TABLE_ROWS, _D_OUT), jnp.float32),
    )(embedding, W1, b1.reshape(1, _D_HID), W2, b2.reshape(1, _D_OUT))


_info = plsc.get_sparse_core_info()
_NC, _NS = _info.num_cores, _info.num_subcores
_NW = _NC * _NS
_BPW = _BATCH // _NW
_CHUNK = 128
_NCHUNK = _BPW // _CHUNK

_sc_mesh = plsc.VectorSubcoreMesh(core_axis_name="c", subcore_axis_name="s")


@functools.partial(
    pl.kernel,
    mesh=_sc_mesh,
    out_type=jax.ShapeDtypeStruct((_BATCH, _D_OUT), jnp.float32),
    scratch_types=[
        pltpu.VMEM((_CHUNK,), jnp.int32),
        pltpu.VMEM((_CHUNK, _D_OUT), jnp.float32),
        pltpu.SemaphoreType.DMA,
    ],
)
def _sc_gather(table_hbm, idx_hbm, out_hbm, idx_v, rows_v, sem):
    wid = lax.axis_index("s") * _NC + lax.axis_index("c")
    base = wid * _BPW
    for c in range(_NCHUNK):
        off = base + c * _CHUNK
        pltpu.sync_copy(idx_hbm.at[pl.ds(off, _CHUNK)], idx_v)
        pltpu.async_copy(table_hbm.at[idx_v], rows_v, sem).wait()
        pltpu.sync_copy(rows_v, out_hbm.at[pl.ds(off, _CHUNK)])


def kernel(diffusion_step, embedding, W1, b1, W2, b2):
    table = _mlp_table(embedding, W1, b1, W2, b2)
    idx = diffusion_step.astype(jnp.int32)
    return _sc_gather(table, idx)


# speedup vs baseline: 1.4797x; 1.0147x over previous
import functools

import jax
import jax.numpy as jnp
from jax import lax
from jax.experimental import pallas as pl
from jax.experimental.pallas import tpu as pltpu
from jax.experimental.pallas import tpu_sc as plsc

_TABLE_ROWS = 1000
_D_IN = 128
_D_HID = 512
_D_OUT = 512
_BATCH = 16384


def _mlp_table_body(emb_ref, w1_ref, b1_ref, w2_ref, b2_ref, out_ref):
    h = jnp.dot(emb_ref[...], w1_ref[...], preferred_element_type=jnp.float32)
    h = h + b1_ref[...]
    h = h * lax.logistic(h)
    o = jnp.dot(h, w2_ref[...], preferred_element_type=jnp.float32)
    o = o + b2_ref[...]
    out_ref[...] = o * lax.logistic(o)


def _mlp_table(embedding, W1, b1, W2, b2):
    return pl.pallas_call(
        _mlp_table_body,
        out_shape=jax.ShapeDtypeStruct((_TABLE_ROWS, _D_OUT), jnp.float32),
    )(embedding, W1, b1.reshape(1, _D_HID), W2, b2.reshape(1, _D_OUT))


_info = plsc.get_sparse_core_info()
_NC, _NS = _info.num_cores, _info.num_subcores
_NW = _NC * _NS
_BPW = _BATCH // _NW
_CHUNK = 64
_NCHUNK = _BPW // _CHUNK
_NBUF = 3

_sc_mesh = plsc.VectorSubcoreMesh(core_axis_name="c", subcore_axis_name="s")


@functools.partial(
    pl.kernel,
    mesh=_sc_mesh,
    out_type=jax.ShapeDtypeStruct((_BATCH, _D_OUT), jnp.float32),
    scratch_types=[
        pltpu.VMEM((_BPW,), jnp.int32),
        pltpu.VMEM((_NBUF, _CHUNK, _D_OUT), jnp.float32),
        pltpu.SemaphoreType.DMA((_NBUF,)),
        pltpu.SemaphoreType.DMA((_NBUF,)),
    ],
)
def _sc_gather(table_hbm, idx_hbm, out_hbm, idx_v, rows_v, gsem, osem):
    wid = lax.axis_index("s") * _NC + lax.axis_index("c")
    base = wid * _BPW
    pltpu.sync_copy(idx_hbm.at[pl.ds(base, _BPW)], idx_v)

    def fire_gather(c):
        return pltpu.async_copy(
            table_hbm.at[idx_v.at[pl.ds(c * _CHUNK, _CHUNK)]],
            rows_v.at[c % _NBUF],
            gsem.at[c % _NBUF],
        )

    def fire_out(c):
        return pltpu.async_copy(
            rows_v.at[c % _NBUF],
            out_hbm.at[pl.ds(base + c * _CHUNK, _CHUNK)],
            osem.at[c % _NBUF],
        )

    gathers = {0: fire_gather(0)}
    outs = {}
    for c in range(_NCHUNK):
        nxt = c + 1
        if nxt < _NCHUNK:
            if nxt - _NBUF in outs:
                outs[nxt - _NBUF].wait()
            gathers[nxt] = fire_gather(nxt)
        gathers[c].wait()
        outs[c] = fire_out(c)
    for c in range(max(0, _NCHUNK - _NBUF), _NCHUNK):
        outs[c].wait()


def kernel(diffusion_step, embedding, W1, b1, W2, b2):
    table = _mlp_table(embedding, W1, b1, W2, b2)
    idx = diffusion_step.astype(jnp.int32)
    return _sc_gather(table, idx)
